# Initial kernel scaffold; baseline (speedup 1.0000x reference)
import jax
import jax.numpy as jnp
from jax.experimental import pallas as pl


def kernel(edge_feat, emb_table, lin_W, lin_b):
    raise NotImplementedError("write your pallas kernel here")


# speedup vs baseline: 5.0073x; 5.0073x over previous
import functools

import jax
import jax.numpy as jnp
from jax import lax
from jax.experimental import pallas as pl
from jax.experimental.pallas import tpu as pltpu
from jax.experimental.pallas import tpu_sc as plsc

E = 320000
F = 4
H = 128
L = 16
NC = 2
NS = 16
NW = NC * NS
EPW = E // NW
B = 400
NCHUNK = EPW // B
GSUB = 80
NGS = B // GSUB


def _body(edge_hbm, table_hbm, w_hbm, b_hbm, out_hbm,
          ec_v, idx0_v, idx1_v, s_v, r0_v, r1_v, w_v, b_v, sem):
    wid = lax.axis_index("s") * NC + lax.axis_index("c")
    base = wid * EPW

    pltpu.sync_copy(w_hbm, w_v)
    pltpu.sync_copy(b_hbm, b_v)
    w_regs = [w_v[pl.ds(h * L, L)] for h in range(H // L)]
    bb_regs = [b_v[pl.ds(h * L, L)] * 2.0 for h in range(H // L)]
    lane = lax.iota(jnp.int32, L)

    def chunk_body(g, carry):
        cbase = base + g * B
        pltpu.sync_copy(edge_hbm.at[pl.ds(cbase * F, B * F)], ec_v)

        def split_body(t, carry):
            pos = (t * L + lane) * F
            i0 = plsc.load_gather(ec_v, [pos])
            i1 = plsc.load_gather(ec_v, [pos + 1])
            c2 = plsc.load_gather(ec_v, [pos + 2])
            c3 = plsc.load_gather(ec_v, [pos + 3])
            sl = pl.ds(t * L, L)
            idx0_v[sl] = i0
            idx1_v[sl] = i1
            s_v[sl] = (c2 + c3).astype(jnp.float32)
            return carry

        lax.fori_loop(0, B // L, split_body, 0)

        copies = []
        for j in range(NGS):
            sl = pl.ds(j * GSUB, GSUB)
            copies.append(pltpu.async_copy(
                table_hbm.at[idx0_v.at[sl]], r0_v.at[sl], sem))
            copies.append(pltpu.async_copy(
                table_hbm.at[idx1_v.at[sl]], r1_v.at[sl], sem))
        for cp in copies:
            cp.wait()

        def edge_body(t, carry):
            s16 = s_v[pl.ds(t * L, L)]
            for i in range(L):
                e = t * L + i
                sv = jnp.full((L,), s16[i], jnp.float32)
                for h in range(H // L):
                    sl = pl.ds(h * L, L)
                    r0_v[e, sl] = (r0_v[e, sl] + r1_v[e, sl]
                                   + sv * w_regs[h] + bb_regs[h])
            return carry

        lax.fori_loop(0, B // L, edge_body, 0)

        pltpu.sync_copy(r0_v, out_hbm.at[pl.ds(cbase, B), :])
        return carry

    lax.fori_loop(0, NCHUNK, chunk_body, 0)


@jax.jit
def _run(edge_flat, emb_table, w_flat, lin_b):
    mesh = plsc.VectorSubcoreMesh(core_axis_name="c", subcore_axis_name="s")
    k = pl.kernel(
        _body,
        out_type=jax.ShapeDtypeStruct((E, H), jnp.float32),
        mesh=mesh,
        compiler_params=pltpu.CompilerParams(needs_layout_passes=False),
        scratch_types=[
            pltpu.VMEM((B * F,), jnp.int32),
            pltpu.VMEM((B,), jnp.int32),
            pltpu.VMEM((B,), jnp.int32),
            pltpu.VMEM((B,), jnp.float32),
            pltpu.VMEM((B, H), jnp.float32),
            pltpu.VMEM((B, H), jnp.float32),
            pltpu.VMEM((H,), jnp.float32),
            pltpu.VMEM((H,), jnp.float32),
            pltpu.SemaphoreType.DMA,
        ],
    )
    return k(edge_flat, emb_table, w_flat, lin_b)


def kernel(edge_feat, emb_table, lin_W, lin_b):
    edge_flat = edge_feat.astype(jnp.int32).reshape(E * F)
    w_flat = lin_W.reshape(H)
    out = _run(edge_flat, emb_table, w_flat, lin_b)
    return out.reshape(1, E, H)


# speedup vs baseline: 6.7343x; 1.3449x over previous
import jax
import jax.numpy as jnp
from jax import lax
from jax.experimental import pallas as pl
from jax.experimental.pallas import tpu as pltpu
from jax.experimental.pallas import tpu_sc as plsc

E = 320000
F = 4
H = 128
L = 16
NC = 2
NS = 16
NW = NC * NS
EPW = E // NW
B = 80
KC = 5
NITER = EPW // (B * KC)


def _body(edge_hbm, table_hbm, w_hbm, b_hbm, out_hbm,
          ec, idx0, idx1, s_v, r0, r1, w_v, b_v, esem, gsem, wsem):
    wid = lax.axis_index("s") * NC + lax.axis_index("c")
    base = wid * EPW

    pltpu.sync_copy(w_hbm, w_v)
    pltpu.sync_copy(b_hbm, b_v)
    w_regs = [w_v[pl.ds(h * L, L)] for h in range(H // L)]
    bb_regs = [b_v[pl.ds(h * L, L)] * 2.0 for h in range(H // L)]
    lane = lax.iota(jnp.int32, L)

    def fire_edge(c, b):
        pltpu.async_copy(edge_hbm.at[pl.ds((base + c * B) * F, B * F)],
                         ec.at[pl.ds(b * B * F, B * F)], esem.at[b])

    def drain(src, dst, sem):
        pltpu.make_async_copy(src, dst, sem).wait()

    for b in range(KC):
        fire_edge(b, b)
        pltpu.async_copy(r0.at[b], out_hbm.at[pl.ds(base + b * B, B), :],
                         wsem.at[b])

    def ring_body(ss, carry):
        c0 = ss * KC
        gcopies = []
        for b in range(KC):
            ecb = ec.at[pl.ds(b * B * F, B * F)]
            drain(edge_hbm.at[pl.ds(base * F, B * F)], ecb, esem.at[b])

            def split_body(t, carry, ecb=ecb, b=b):
                pos = (t * L + lane) * F
                i0 = plsc.load_gather(ecb, [pos])
                i1 = plsc.load_gather(ecb, [pos + 1])
                c2 = plsc.load_gather(ecb, [pos + 2])
                c3 = plsc.load_gather(ecb, [pos + 3])
                sl = pl.ds(b * B + t * L, L)
                idx0[sl] = i0
                idx1[sl] = i1
                s_v[sl] = (c2 + c3).astype(jnp.float32)
                return carry

            lax.fori_loop(0, B // L, split_body, 0)
            drain(out_hbm.at[pl.ds(base, B), :], r0.at[b], wsem.at[b])
            gcopies.append(pltpu.async_copy(
                table_hbm.at[idx0.at[pl.ds(b * B, B)]], r0.at[b],
                gsem.at[b]))
            gcopies.append(pltpu.async_copy(
                table_hbm.at[idx1.at[pl.ds(b * B, B)]], r1.at[b],
                gsem.at[b]))

        @pl.when(ss < NITER - 1)
        def _():
            for b in range(KC):
                fire_edge(c0 + KC + b, b)

        for b in range(KC):
            gcopies[2 * b].wait()
            gcopies[2 * b + 1].wait()
            r0b, r1b = r0.at[b], r1.at[b]
            svb = s_v.at[pl.ds(b * B, B)]

            def edge_body(e, carry, r0b=r0b, r1b=r1b, svb=svb):
                sv = plsc.load_gather(svb, [jnp.full((L,), e, jnp.int32)])
                for h in range(H // L):
                    sl = pl.ds(h * L, L)
                    r0b[e, sl] = (r0b[e, sl] + r1b[e, sl]
                                  + sv * w_regs[h] + bb_regs[h])
                return carry

            lax.fori_loop(0, B, edge_body, 0)
            pltpu.async_copy(
                r0.at[b], out_hbm.at[pl.ds(base + (c0 + b) * B, B), :],
                wsem.at[b])
        return carry

    lax.fori_loop(0, NITER, ring_body, 0)

    for b in range(KC):
        drain(out_hbm.at[pl.ds(base, B), :], r0.at[b], wsem.at[b])


@jax.jit
def _run(edge_flat, emb_table, w_flat, lin_b):
    mesh = plsc.VectorSubcoreMesh(core_axis_name="c", subcore_axis_name="s")
    k = pl.kernel(
        _body,
        out_type=jax.ShapeDtypeStruct((E, H), jnp.float32),
        mesh=mesh,
        compiler_params=pltpu.CompilerParams(needs_layout_passes=False),
        scratch_types=[
            pltpu.VMEM((KC * B * F,), jnp.int32),
            pltpu.VMEM((KC * B,), jnp.int32),
            pltpu.VMEM((KC * B,), jnp.int32),
            pltpu.VMEM((KC * B,), jnp.float32),
            pltpu.VMEM((KC, B, H), jnp.float32),
            pltpu.VMEM((KC, B, H), jnp.float32),
            pltpu.VMEM((H,), jnp.float32),
            pltpu.VMEM((H,), jnp.float32),
            pltpu.SemaphoreType.DMA((KC,)),
            pltpu.SemaphoreType.DMA((KC,)),
            pltpu.SemaphoreType.DMA((KC,)),
        ],
    )
    return k(edge_flat, emb_table, w_flat, lin_b)


def kernel(edge_feat, emb_table, lin_W, lin_b):
    edge_flat = edge_feat.astype(jnp.int32).reshape(E * F)
    w_flat = lin_W.reshape(H)
    out = _run(edge_flat, emb_table, w_flat, lin_b)
    return out.reshape(1, E, H)


# speedup vs baseline: 9.5693x; 1.4210x over previous
import jax
import jax.numpy as jnp
from jax import lax
from jax.experimental import pallas as pl
from jax.experimental.pallas import tpu as pltpu
from jax.experimental.pallas import tpu_sc as plsc

E = 320000
F = 4
H = 128
L = 16
NC = 2
NS = 16
NW = NC * NS
EPW = E // NW
B = 80
KC = 5
NITER = EPW // (B * KC)


def _body(edge_hbm, table_hbm, w_hbm, b_hbm, out_hbm,
          ec, idx0, idx1, s_v, r0, r1, w_v, b_v, esem, gsem, wsem):
    wid = lax.axis_index("s") * NC + lax.axis_index("c")
    base = wid * EPW

    pltpu.sync_copy(w_hbm, w_v)
    pltpu.sync_copy(b_hbm, b_v)
    w_regs = [w_v[pl.ds(h * L, L)] for h in range(H // L)]
    bb_regs = [b_v[pl.ds(h * L, L)] * 2.0 for h in range(H // L)]
    lane = lax.iota(jnp.int32, L)

    def fire_edge(c, b):
        pltpu.async_copy(edge_hbm.at[pl.ds((base + c * B) * F, B * F)],
                         ec.at[pl.ds(b * B * F, B * F)], esem.at[b])

    def drain(src, dst, sem):
        pltpu.make_async_copy(src, dst, sem).wait()

    for b in range(KC):
        fire_edge(b, b)
        pltpu.async_copy(r0.at[b], out_hbm.at[pl.ds(base + b * B, B), :],
                         wsem.at[b])

    def ring_body(ss, carry):
        c0 = ss * KC
        gcopies = []
        for b in range(KC):
            ecb = ec.at[pl.ds(b * B * F, B * F)]
            drain(edge_hbm.at[pl.ds(base * F, B * F)], ecb, esem.at[b])

            @plsc.parallel_loop(0, B // L, 1, unroll=2)
            def split_body(t, ecb=ecb, b=b):
                pos = (t * L + lane) * F
                i0 = plsc.load_gather(ecb, [pos])
                i1 = plsc.load_gather(ecb, [pos + 1])
                c2 = plsc.load_gather(ecb, [pos + 2])
                c3 = plsc.load_gather(ecb, [pos + 3])
                sl = pl.ds(b * B + t * L, L)
                idx0[sl] = i0
                idx1[sl] = i1
                s_v[sl] = (c2 + c3).astype(jnp.float32)
            drain(out_hbm.at[pl.ds(base, B), :], r0.at[b], wsem.at[b])
            gcopies.append(pltpu.async_copy(
                table_hbm.at[idx0.at[pl.ds(b * B, B)]], r0.at[b],
                gsem.at[b]))
            gcopies.append(pltpu.async_copy(
                table_hbm.at[idx1.at[pl.ds(b * B, B)]], r1.at[b],
                gsem.at[b]))

        @pl.when(ss < NITER - 1)
        def _():
            for b in range(KC):
                fire_edge(c0 + KC + b, b)

        for b in range(KC):
            gcopies[2 * b].wait()
            gcopies[2 * b + 1].wait()
            r0b, r1b = r0.at[b], r1.at[b]
            svb = s_v.at[pl.ds(b * B, B)]

            @plsc.parallel_loop(0, B, 1, unroll=2)
            def edge_body(e, r0b=r0b, r1b=r1b, svb=svb):
                sv = plsc.load_gather(svb, [jnp.full((L,), e, jnp.int32)])
                for h in range(H // L):
                    sl = pl.ds(h * L, L)
                    r0b[e, sl] = (r0b[e, sl] + r1b[e, sl]
                                  + sv * w_regs[h] + bb_regs[h])
            pltpu.async_copy(
                r0.at[b], out_hbm.at[pl.ds(base + (c0 + b) * B, B), :],
                wsem.at[b])
        return carry

    lax.fori_loop(0, NITER, ring_body, 0)

    for b in range(KC):
        drain(out_hbm.at[pl.ds(base, B), :], r0.at[b], wsem.at[b])


@jax.jit
def _run(edge_flat, emb_table, w_flat, lin_b):
    mesh = plsc.VectorSubcoreMesh(core_axis_name="c", subcore_axis_name="s")
    k = pl.kernel(
        _body,
        out_type=jax.ShapeDtypeStruct((E, H), jnp.float32),
        mesh=mesh,
        compiler_params=pltpu.CompilerParams(needs_layout_passes=False),
        scratch_types=[
            pltpu.VMEM((KC * B * F,), jnp.int32),
            pltpu.VMEM((KC * B,), jnp.int32),
            pltpu.VMEM((KC * B,), jnp.int32),
            pltpu.VMEM((KC * B,), jnp.float32),
            pltpu.VMEM((KC, B, H), jnp.float32),
            pltpu.VMEM((KC, B, H), jnp.float32),
            pltpu.VMEM((H,), jnp.float32),
            pltpu.VMEM((H,), jnp.float32),
            pltpu.SemaphoreType.DMA((KC,)),
            pltpu.SemaphoreType.DMA((KC,)),
            pltpu.SemaphoreType.DMA((KC,)),
        ],
    )
    return k(edge_flat, emb_table, w_flat, lin_b)


def kernel(edge_feat, emb_table, lin_W, lin_b):
    edge_flat = edge_feat.astype(jnp.int32).reshape(E * F)
    w_flat = lin_W.reshape(H)
    out = _run(edge_flat, emb_table, w_flat, lin_b)
    return out.reshape(1, E, H)
